# baseline (speedup 1.0000x reference)
import functools
import math

import jax
import jax.numpy as jnp
from jax import lax
from jax.experimental import pallas as pl
from jax.experimental.pallas import tpu as pltpu
from jax.experimental.pallas import tpu_sc as plsc

HIDDEN = 128
CUTOFF = 5.0
GAUSS_STEP = 0.1
LOG2 = math.log(2.0)

NC = 2
NS = 16
NW = NC * NS
CH = 128


def _mesh():
  return plsc.VectorSubcoreMesh(
      core_axis_name="c", subcore_axis_name="s", num_cores=NC, num_subcores=NS)


def _ssp(x):
  return jnp.log(0.5 + 0.5 * jnp.exp(x))


def _vadd(a_buf, b_buf, o_buf, rows):
  def rbody(r, carry):
    for j in range(HIDDEN // 16):
      sl = pl.ds(j * 16, 16)
      o_buf[r, sl] = a_buf[r, sl] + b_buf[r, sl]
    return carry

  lax.fori_loop(0, rows, rbody, 0)


def _sc_gather_add(table_a, table_b, idx_a, idx_b):
  n = idx_a.shape[0]
  d = table_a.shape[1]
  per_w = n // NW
  nch = per_w // CH
  rem = per_w % CH
  nchunks = nch + (1 if rem else 0)
  nfp = (nch - 2) // 2
  assert nch >= 6 and per_w % 8 == 0

  scratch = [
      pltpu.VMEM((per_w,), jnp.int32),
      pltpu.VMEM((per_w,), jnp.int32),
      pltpu.VMEM((CH, d), jnp.float32),
      pltpu.VMEM((CH, d), jnp.float32),
      pltpu.VMEM((CH, d), jnp.float32),
      pltpu.VMEM((CH, d), jnp.float32),
      pltpu.VMEM((CH, d), jnp.float32),
      pltpu.VMEM((CH, d), jnp.float32),
      pltpu.SemaphoreType.DMA,
      pltpu.SemaphoreType.DMA,
      pltpu.SemaphoreType.DMA,
      pltpu.SemaphoreType.DMA,
      pltpu.SemaphoreType.DMA,
      pltpu.SemaphoreType.DMA,
  ]

  @functools.partial(
      pl.kernel,
      mesh=_mesh(),
      out_type=jax.ShapeDtypeStruct((n, d), jnp.float32),
      scratch_types=scratch,
  )
  def k(ta, tb, ia, ib, out, ia_all, ib_all, a0, b0, o0, a1, b1, o1,
        ga0, gb0, go0, ga1, gb1, go1):
    abuf = (a0, a1)
    bbuf = (b0, b1)
    obuf = (o0, o1)
    gas = (ga0, ga1)
    gbs = (gb0, gb1)
    gos = (go0, go1)
    wid = lax.axis_index("s") * NC + lax.axis_index("c")
    base = wid * per_w

    pltpu.sync_copy(ia.at[pl.ds(base, per_w)], ia_all)
    pltpu.sync_copy(ib.at[pl.ds(base, per_w)], ib_all)

    def size_of(c):
      return CH if c < nch else rem

    def issue_gather(c, b, sz=CH):
      off = c * CH
      pltpu.async_copy(ta.at[ia_all.at[pl.ds(off, sz)]],
                       abuf[b].at[pl.ds(0, sz)], gas[b])
      pltpu.async_copy(tb.at[ib_all.at[pl.ds(off, sz)]],
                       bbuf[b].at[pl.ds(0, sz)], gbs[b])

    def wait_gather(b, sz=CH):
      pltpu.make_async_copy(ta.at[pl.ds(0, sz)], abuf[b].at[pl.ds(0, sz)],
                            gas[b]).wait()
      pltpu.make_async_copy(tb.at[pl.ds(0, sz)], bbuf[b].at[pl.ds(0, sz)],
                            gbs[b]).wait()

    def issue_out(c, b, sz=CH):
      pltpu.async_copy(obuf[b].at[pl.ds(0, sz)],
                       out.at[pl.ds(base + c * CH, sz)], gos[b])

    def wait_out(b, sz=CH):
      pltpu.make_async_copy(obuf[b].at[pl.ds(0, sz)],
                            out.at[pl.ds(0, sz)], gos[b]).wait()

    issue_gather(0, 0)
    issue_gather(1, 1)
    for b in (0, 1):
      wait_gather(b)
      _vadd(abuf[b], bbuf[b], obuf[b], CH)
      issue_out(b, b)
      issue_gather(b + 2, b)

    def body(i, carry):
      c0 = 2 * i
      for b in (0, 1):
        c = c0 + b
        wait_gather(b)
        wait_out(b)
        _vadd(abuf[b], bbuf[b], obuf[b], CH)
        issue_out(c, b)
        issue_gather(c + 2, b)
      return carry

    lax.fori_loop(1, nfp, body, 0)

    for c in range(2 * nfp, nchunks):
      b = c & 1
      sz = size_of(c)
      wait_gather(b, sz)
      wait_out(b)
      _vadd(abuf[b], bbuf[b], obuf[b], sz)
      issue_out(c, b, sz)
      if c + 2 < nchunks:
        issue_gather(c + 2, b, size_of(c + 2))

    wait_out((nchunks - 2) & 1, size_of(nchunks - 2))
    wait_out((nchunks - 1) & 1, size_of(nchunks - 1))

  return k(table_a, table_b, idx_a, idx_b)


def _sc_scatter(m_arr, dst_arr, n_nodes_pad):
  e = m_arr.shape[0]
  d = m_arr.shape[1]
  per_sc = e // NC
  per_w = per_sc // NS
  nch = per_w // CH
  rem = per_w % CH
  nchunks = nch + (1 if rem else 0)
  nfp = (nch - 2) // 2
  assert nch >= 6 and per_w % 8 == 0
  rows_per_sub = n_nodes_pad // NS
  nz = rows_per_sub // CH

  scratch = [
      pltpu.VMEM((2, CH), jnp.int32),
      pltpu.VMEM((CH, d), jnp.float32),
      pltpu.VMEM((CH, d), jnp.float32),
      pltpu.VMEM_SHARED((n_nodes_pad, d), jnp.float32),
      pltpu.SemaphoreType.DMA,
      pltpu.SemaphoreType.DMA,
  ]

  @functools.partial(
      pl.kernel,
      mesh=_mesh(),
      out_type=jax.ShapeDtypeStruct((NC, n_nodes_pad, d), jnp.float32),
      scratch_types=scratch,
  )
  def k(m_hbm, dst_hbm, out_hbm, idx2, m0, m1, acc, f0, f1):
    mbuf = (m0, m1)
    fs = (f0, f1)
    cid = lax.axis_index("c")
    sid = lax.axis_index("s")

    zeros16 = jnp.zeros((16,), jnp.float32)

    def zbody(i, carry):
      for j in range(d // 16):
        m0[i, pl.ds(j * 16, 16)] = zeros16
      return carry

    lax.fori_loop(0, CH, zbody, 0)
    row0 = sid * rows_per_sub
    for kk in range(nz):
      pltpu.sync_copy(m0.at[pl.ds(0, CH)], acc.at[pl.ds(row0 + kk * CH, CH)])
    plsc.subcore_barrier()

    base = cid * per_sc + sid * per_w

    if True:
      def issue_fetch(c, b, sz=CH):
        off = base + c * CH
        pltpu.async_copy(dst_hbm.at[pl.ds(off, sz)], idx2.at[b, pl.ds(0, sz)],
                         fs[b])
        pltpu.async_copy(m_hbm.at[pl.ds(off, sz)], mbuf[b].at[pl.ds(0, sz)],
                         fs[b])

      def wait_fetch(b, sz=CH):
        pltpu.make_async_copy(dst_hbm.at[pl.ds(0, sz)],
                              idx2.at[b, pl.ds(0, sz)], fs[b]).wait()
        pltpu.make_async_copy(m_hbm.at[pl.ds(0, sz)],
                              mbuf[b].at[pl.ds(0, sz)], fs[b]).wait()

      def scat(b, sz=CH):
        if sz == CH:
          pltpu.sync_copy(mbuf[b], acc.at[idx2.at[b]], add=True)
        else:
          pltpu.sync_copy(mbuf[b].at[pl.ds(0, sz)],
                          acc.at[idx2.at[b, pl.ds(0, sz)]], add=True)

      def size_of(c):
        return CH if c < nch else rem

      issue_fetch(0, 0)
      issue_fetch(1, 1)

      def body(i, carry):
        c0 = 2 * i
        for b in (0, 1):
          wait_fetch(b)
          scat(b)
          issue_fetch(c0 + b + 2, b)
        return carry

      lax.fori_loop(0, nfp, body, 0)
      for c in range(2 * nfp, nchunks):
        b = c & 1
        sz = size_of(c)
        wait_fetch(b, sz)
        scat(b, sz)
        if c + 2 < nchunks:
          issue_fetch(c + 2, b, size_of(c + 2))

    plsc.subcore_barrier()

    for kk in range(nz):
      r = row0 + kk * CH
      pltpu.sync_copy(acc.at[pl.ds(r, CH)], m0.at[pl.ds(0, CH)])
      pltpu.sync_copy(m0.at[pl.ds(0, CH)], out_hbm.at[cid, pl.ds(r, CH)])

  return k(m_arr, dst_arr)


def _tc_embed_pre(node_idx, emb_pad, wa, wb, b1):
  n = node_idx.shape[0]
  t = 2000
  grid = n // t

  def body(idx_ref, emb_ref, wa_ref, wb_ref, b_ref, ns_ref, a_ref, bm_ref):
    cols = lax.broadcasted_iota(jnp.int32, (1, HIDDEN), 1)
    onehot = (idx_ref[...] == cols).astype(jnp.float32)
    ns = jnp.dot(onehot, emb_ref[...], preferred_element_type=jnp.float32)
    ns_ref[...] = ns
    a_ref[...] = jnp.dot(ns, wa_ref[...], preferred_element_type=jnp.float32)
    bm_ref[...] = jnp.dot(ns, wb_ref[...],
                          preferred_element_type=jnp.float32) + b_ref[...]

  return pl.pallas_call(
      body,
      grid=(grid,),
      in_specs=[
          pl.BlockSpec((t, 1), lambda i: (i, 0)),
          pl.BlockSpec((HIDDEN, HIDDEN), lambda i: (0, 0)),
          pl.BlockSpec((HIDDEN, HIDDEN), lambda i: (0, 0)),
          pl.BlockSpec((HIDDEN, HIDDEN), lambda i: (0, 0)),
          pl.BlockSpec((1, HIDDEN), lambda i: (0, 0)),
      ],
      out_specs=(pl.BlockSpec((t, HIDDEN), lambda i: (i, 0)),
                 pl.BlockSpec((t, HIDDEN), lambda i: (i, 0)),
                 pl.BlockSpec((t, HIDDEN), lambda i: (i, 0))),
      out_shape=(jax.ShapeDtypeStruct((n, HIDDEN), jnp.float32),
                 jax.ShapeDtypeStruct((n, HIDDEN), jnp.float32),
                 jax.ShapeDtypeStruct((n, HIDDEN), jnp.float32)),
  )(node_idx.reshape(n, 1), emb_pad, wa, wb, b1.reshape(1, HIDDEN))


def _tc_edge(g, feat, we_pad, be, w2, b2, sc=None):
  e = g.shape[0]
  t = next(tt for tt in (2048, 1024, 512, 256) if e % tt == 0)
  grid = e // t
  inv2s2 = 1.0 / (2.0 * GAUSS_STEP * GAUSS_STEP)
  first = sc is None

  def body(g_ref, f_ref, we_ref, be_ref, w2_ref, b2_ref, *rest):
    if first:
      m_ref, sc_ref = rest
    else:
      sc_in, m_ref = rest
    x = f_ref[...]
    mu = lax.broadcasted_iota(jnp.int32, (1, HIDDEN), 1).astype(
        jnp.float32) * GAUSS_STEP
    ex = jnp.exp(-((x - mu) ** 2) * inv2s2)
    if first:
      cut = 1.0 / (1.0 + jnp.exp(5.0 * (x - (CUTOFF - 1.5))))
      sc_ref[...] = cut
    else:
      cut = sc_in[...]
    gate = _ssp(jnp.dot(ex, we_ref[...], preferred_element_type=jnp.float32)
                + be_ref[...]) * cut
    h = _ssp(g_ref[...])
    m_ref[...] = (jnp.dot(h, w2_ref[...], preferred_element_type=jnp.float32)
                  + b2_ref[...]) * gate

  in_specs = [
      pl.BlockSpec((t, HIDDEN), lambda i: (i, 0)),
      pl.BlockSpec((t, 1), lambda i: (i, 0)),
      pl.BlockSpec((HIDDEN, HIDDEN), lambda i: (0, 0)),
      pl.BlockSpec((1, HIDDEN), lambda i: (0, 0)),
      pl.BlockSpec((HIDDEN, HIDDEN), lambda i: (0, 0)),
      pl.BlockSpec((1, HIDDEN), lambda i: (0, 0)),
  ]
  args = [g, feat, we_pad, be.reshape(1, HIDDEN), w2, b2.reshape(1, HIDDEN)]
  m_spec = pl.BlockSpec((t, HIDDEN), lambda i: (i, 0))
  m_shape = jax.ShapeDtypeStruct((e, HIDDEN), jnp.float32)
  sc_spec = pl.BlockSpec((t, 1), lambda i: (i, 0))
  if first:
    out_specs = (m_spec, sc_spec)
    out_shape = (m_shape, jax.ShapeDtypeStruct((e, 1), jnp.float32))
  else:
    in_specs.append(sc_spec)
    args.append(sc)
    out_specs = m_spec
    out_shape = m_shape

  return pl.pallas_call(
      body,
      grid=(grid,),
      in_specs=in_specs,
      out_specs=out_specs,
      out_shape=out_shape,
  )(*args)


def _tc_node_post(ns, plist, w1, b1, w2, b2, nxt=None):
  n = ns.shape[0]
  t = 2000
  grid = n // t
  fused = nxt is not None
  np_ = len(plist)

  def body(ns_ref, *rest):
    p_refs = rest[:np_]
    w1_ref, b1_ref, w2_ref, b2_ref = rest[np_:np_ + 4]
    rest = rest[np_ + 4:]
    if fused:
      wa_ref, wb_ref, nb_ref, o_ref, a_ref, bm_ref = rest
    else:
      (o_ref,) = rest
    ms = p_refs[0][...]
    for pr in p_refs[1:]:
      ms = ms + pr[...]
    tt = _ssp(jnp.dot(ms, w1_ref[...], preferred_element_type=jnp.float32)
              + b1_ref[...])
    new = ns_ref[...] + jnp.dot(
        tt, w2_ref[...], preferred_element_type=jnp.float32) + b2_ref[...]
    o_ref[...] = new
    if fused:
      a_ref[...] = jnp.dot(new, wa_ref[...], preferred_element_type=jnp.float32)
      bm_ref[...] = jnp.dot(new, wb_ref[...],
                            preferred_element_type=jnp.float32) + nb_ref[...]

  in_specs = [pl.BlockSpec((t, HIDDEN), lambda i: (i, 0))]
  in_specs += [pl.BlockSpec((t, HIDDEN), lambda i: (i, 0))] * np_
  in_specs += [
      pl.BlockSpec((HIDDEN, HIDDEN), lambda i: (0, 0)),
      pl.BlockSpec((1, HIDDEN), lambda i: (0, 0)),
      pl.BlockSpec((HIDDEN, HIDDEN), lambda i: (0, 0)),
      pl.BlockSpec((1, HIDDEN), lambda i: (0, 0)),
  ]
  args = [ns] + list(plist) + [w1, b1.reshape(1, HIDDEN), w2,
                               b2.reshape(1, HIDDEN)]
  out_specs = pl.BlockSpec((t, HIDDEN), lambda i: (i, 0))
  out_shape = jax.ShapeDtypeStruct((n, HIDDEN), jnp.float32)
  if fused:
    wa, wb, nb1 = nxt
    in_specs += [
        pl.BlockSpec((HIDDEN, HIDDEN), lambda i: (0, 0)),
        pl.BlockSpec((HIDDEN, HIDDEN), lambda i: (0, 0)),
        pl.BlockSpec((1, HIDDEN), lambda i: (0, 0)),
    ]
    args += [wa, wb, nb1.reshape(1, HIDDEN)]
    out_specs = (out_specs,) * 3
    out_shape = (out_shape,) * 3

  return pl.pallas_call(
      body,
      grid=(grid,),
      in_specs=in_specs,
      out_specs=out_specs,
      out_shape=out_shape,
  )(*args)


def kernel(nodes, num_nodes, atom_edges, num_atom_edges, atom_edges_features,
           atom_embeddings, params):
  del num_nodes, num_atom_edges
  n_nodes = nodes.shape[1]
  node_idx = nodes[0].astype(jnp.int32)
  src = atom_edges[0, :, 0].astype(jnp.int32)
  dst = atom_edges[0, :, 1].astype(jnp.int32)
  feat = atom_edges_features[0].astype(jnp.float32)

  n_pad = ((n_nodes + CH * NS - 1) // (CH * NS)) * (CH * NS)
  emb_pad = jnp.zeros((HIDDEN, HIDDEN), jnp.float32).at[
      :atom_embeddings.shape[0]].set(atom_embeddings.astype(jnp.float32))

  e = src.shape[0]
  nsplit = 3
  unit = e // (256 * nsplit) * 256
  offs = [0] + [unit * (h + 1) for h in range(nsplit - 1)] + [e]
  srcs = [src[offs[h]:offs[h + 1]] for h in range(nsplit)]
  dsts = [dst[offs[h]:offs[h + 1]] for h in range(nsplit)]
  feats = [feat[offs[h]:offs[h + 1]] for h in range(nsplit)]

  nl = len(params)
  p = params[0]
  ns, a, bm = _tc_embed_pre(node_idx, emb_pad, p['W_node1'][:HIDDEN],
                            p['W_node1'][HIDDEN:], p['b_node1'])
  soft_cut = [None] * nsplit
  outs = []
  for li in range(nl):
    p = params[li]
    we_pad = jnp.zeros((HIDDEN, HIDDEN), jnp.float32).at[
        :p['W_edge'].shape[0]].set(p['W_edge'])
    g = [_sc_gather_add(a, bm, srcs[h], dsts[h]) for h in range(nsplit)]
    parts = []
    for h in range(nsplit):
      if soft_cut[h] is None:
        m, soft_cut[h] = _tc_edge(g[h], feats[h], we_pad, p['b_edge'],
                                  p['W_node2'], p['b_node2'])
      else:
        m = _tc_edge(g[h], feats[h], we_pad, p['b_edge'], p['W_node2'],
                     p['b_node2'], sc=soft_cut[h])
      parts.append(_sc_scatter(m, dsts[h], n_pad))
    plist = [pp[c, :n_nodes] for pp in parts for c in range(NC)]
    if li + 1 < nl:
      pn = params[li + 1]
      ns, a, bm = _tc_node_post(
          ns, plist, p['W_st1'], p['b_st1'], p['W_st2'], p['b_st2'],
          nxt=(pn['W_node1'][:HIDDEN], pn['W_node1'][HIDDEN:], pn['b_node1']))
    else:
      ns = _tc_node_post(ns, plist, p['W_st1'], p['b_st1'], p['W_st2'],
                         p['b_st2'])
    outs.append(ns)
  return jnp.stack(outs, axis=0)


# speedup vs baseline: 1.0923x; 1.0923x over previous
import functools
import math

import jax
import jax.numpy as jnp
from jax import lax
from jax.experimental import pallas as pl
from jax.experimental.pallas import tpu as pltpu
from jax.experimental.pallas import tpu_sc as plsc

HIDDEN = 128
CUTOFF = 5.0
GAUSS_STEP = 0.1
LOG2 = math.log(2.0)

NC = 2
NS = 16
NW = NC * NS
CH = 128


def _mesh():
  return plsc.VectorSubcoreMesh(
      core_axis_name="c", subcore_axis_name="s", num_cores=NC, num_subcores=NS)


def _ssp(x):
  return jnp.log(0.5 + 0.5 * jnp.exp(x))


def _vadd(a_buf, b_buf, o_buf, rows):
  def rbody(r2, carry):
    r = r2 * 2
    for rr in (0, 1):
      for j in range(HIDDEN // 16):
        sl = pl.ds(j * 16, 16)
        o_buf[r + rr, sl] = a_buf[r + rr, sl] + b_buf[r + rr, sl]
    return carry

  lax.fori_loop(0, rows // 2, rbody, 0)
  if rows % 2:
    r = rows - 1
    for j in range(HIDDEN // 16):
      sl = pl.ds(j * 16, 16)
      o_buf[r, sl] = a_buf[r, sl] + b_buf[r, sl]


def _sc_gather_add(table_a, table_b, idx_a, idx_b):
  n = idx_a.shape[0]
  d = table_a.shape[1]
  per_w = n // NW
  nch = per_w // CH
  rem = per_w % CH
  nchunks = nch + (1 if rem else 0)
  nfp = (nch - 2) // 2
  assert nch >= 6 and per_w % 8 == 0

  scratch = [
      pltpu.VMEM((per_w,), jnp.int32),
      pltpu.VMEM((per_w,), jnp.int32),
      pltpu.VMEM((CH, d), jnp.float32),
      pltpu.VMEM((CH, d), jnp.float32),
      pltpu.VMEM((CH, d), jnp.float32),
      pltpu.VMEM((CH, d), jnp.float32),
      pltpu.VMEM((CH, d), jnp.float32),
      pltpu.VMEM((CH, d), jnp.float32),
      pltpu.SemaphoreType.DMA,
      pltpu.SemaphoreType.DMA,
      pltpu.SemaphoreType.DMA,
      pltpu.SemaphoreType.DMA,
      pltpu.SemaphoreType.DMA,
      pltpu.SemaphoreType.DMA,
  ]

  @functools.partial(
      pl.kernel,
      mesh=_mesh(),
      out_type=jax.ShapeDtypeStruct((n, d), jnp.float32),
      scratch_types=scratch,
  )
  def k(ta, tb, ia, ib, out, ia_all, ib_all, a0, b0, o0, a1, b1, o1,
        ga0, gb0, go0, ga1, gb1, go1):
    abuf = (a0, a1)
    bbuf = (b0, b1)
    obuf = (o0, o1)
    gas = (ga0, ga1)
    gbs = (gb0, gb1)
    gos = (go0, go1)
    wid = lax.axis_index("s") * NC + lax.axis_index("c")
    base = wid * per_w

    pltpu.sync_copy(ia.at[pl.ds(base, per_w)], ia_all)
    pltpu.sync_copy(ib.at[pl.ds(base, per_w)], ib_all)

    def size_of(c):
      return CH if c < nch else rem

    def issue_gather(c, b, sz=CH):
      off = c * CH
      pltpu.async_copy(ta.at[ia_all.at[pl.ds(off, sz)]],
                       abuf[b].at[pl.ds(0, sz)], gas[b])
      pltpu.async_copy(tb.at[ib_all.at[pl.ds(off, sz)]],
                       bbuf[b].at[pl.ds(0, sz)], gbs[b])

    def wait_gather(b, sz=CH):
      pltpu.make_async_copy(ta.at[pl.ds(0, sz)], abuf[b].at[pl.ds(0, sz)],
                            gas[b]).wait()
      pltpu.make_async_copy(tb.at[pl.ds(0, sz)], bbuf[b].at[pl.ds(0, sz)],
                            gbs[b]).wait()

    def issue_out(c, b, sz=CH):
      pltpu.async_copy(obuf[b].at[pl.ds(0, sz)],
                       out.at[pl.ds(base + c * CH, sz)], gos[b])

    def wait_out(b, sz=CH):
      pltpu.make_async_copy(obuf[b].at[pl.ds(0, sz)],
                            out.at[pl.ds(0, sz)], gos[b]).wait()

    issue_gather(0, 0)
    issue_gather(1, 1)
    for b in (0, 1):
      wait_gather(b)
      _vadd(abuf[b], bbuf[b], obuf[b], CH)
      issue_out(b, b)
      issue_gather(b + 2, b)

    def body(i, carry):
      c0 = 2 * i
      for b in (0, 1):
        c = c0 + b
        wait_gather(b)
        wait_out(b)
        _vadd(abuf[b], bbuf[b], obuf[b], CH)
        issue_out(c, b)
        issue_gather(c + 2, b)
      return carry

    lax.fori_loop(1, nfp, body, 0)

    for c in range(2 * nfp, nchunks):
      b = c & 1
      sz = size_of(c)
      wait_gather(b, sz)
      wait_out(b)
      _vadd(abuf[b], bbuf[b], obuf[b], sz)
      issue_out(c, b, sz)
      if c + 2 < nchunks:
        issue_gather(c + 2, b, size_of(c + 2))

    wait_out((nchunks - 2) & 1, size_of(nchunks - 2))
    wait_out((nchunks - 1) & 1, size_of(nchunks - 1))

  return k(table_a, table_b, idx_a, idx_b)


def _sc_scatter(m_arr, dst_arr, n_nodes_pad):
  e = m_arr.shape[0]
  d = m_arr.shape[1]
  per_sc = e // NC
  per_w = per_sc // NS
  nch = per_w // CH
  rem = per_w % CH
  nchunks = nch + (1 if rem else 0)
  nfp = (nch - 2) // 2
  assert nch >= 6 and per_w % 8 == 0
  rows_per_sub = n_nodes_pad // NS
  nz = rows_per_sub // CH

  scratch = [
      pltpu.VMEM((2, CH), jnp.int32),
      pltpu.VMEM((CH, d), jnp.float32),
      pltpu.VMEM((CH, d), jnp.float32),
      pltpu.VMEM_SHARED((n_nodes_pad, d), jnp.float32),
      pltpu.SemaphoreType.DMA,
      pltpu.SemaphoreType.DMA,
  ]

  @functools.partial(
      pl.kernel,
      mesh=_mesh(),
      out_type=jax.ShapeDtypeStruct((NC, n_nodes_pad, d), jnp.float32),
      scratch_types=scratch,
  )
  def k(m_hbm, dst_hbm, out_hbm, idx2, m0, m1, acc, f0, f1):
    mbuf = (m0, m1)
    fs = (f0, f1)
    cid = lax.axis_index("c")
    sid = lax.axis_index("s")

    zeros16 = jnp.zeros((16,), jnp.float32)

    def zbody(i, carry):
      for j in range(d // 16):
        m0[i, pl.ds(j * 16, 16)] = zeros16
      return carry

    lax.fori_loop(0, CH, zbody, 0)
    row0 = sid * rows_per_sub
    for kk in range(nz):
      pltpu.sync_copy(m0.at[pl.ds(0, CH)], acc.at[pl.ds(row0 + kk * CH, CH)])
    plsc.subcore_barrier()

    base = cid * per_sc + sid * per_w

    if True:
      def issue_fetch(c, b, sz=CH):
        off = base + c * CH
        pltpu.async_copy(dst_hbm.at[pl.ds(off, sz)], idx2.at[b, pl.ds(0, sz)],
                         fs[b])
        pltpu.async_copy(m_hbm.at[pl.ds(off, sz)], mbuf[b].at[pl.ds(0, sz)],
                         fs[b])

      def wait_fetch(b, sz=CH):
        pltpu.make_async_copy(dst_hbm.at[pl.ds(0, sz)],
                              idx2.at[b, pl.ds(0, sz)], fs[b]).wait()
        pltpu.make_async_copy(m_hbm.at[pl.ds(0, sz)],
                              mbuf[b].at[pl.ds(0, sz)], fs[b]).wait()

      def scat(b, sz=CH):
        if sz == CH:
          pltpu.sync_copy(mbuf[b], acc.at[idx2.at[b]], add=True)
        else:
          pltpu.sync_copy(mbuf[b].at[pl.ds(0, sz)],
                          acc.at[idx2.at[b, pl.ds(0, sz)]], add=True)

      def size_of(c):
        return CH if c < nch else rem

      issue_fetch(0, 0)
      issue_fetch(1, 1)

      def body(i, carry):
        c0 = 2 * i
        for b in (0, 1):
          wait_fetch(b)
          scat(b)
          issue_fetch(c0 + b + 2, b)
        return carry

      lax.fori_loop(0, nfp, body, 0)
      for c in range(2 * nfp, nchunks):
        b = c & 1
        sz = size_of(c)
        wait_fetch(b, sz)
        scat(b, sz)
        if c + 2 < nchunks:
          issue_fetch(c + 2, b, size_of(c + 2))

    plsc.subcore_barrier()

    for kk in range(nz):
      r = row0 + kk * CH
      pltpu.sync_copy(acc.at[pl.ds(r, CH)], m0.at[pl.ds(0, CH)])
      pltpu.sync_copy(m0.at[pl.ds(0, CH)], out_hbm.at[cid, pl.ds(r, CH)])

  return k(m_arr, dst_arr)


def _tc_embed_pre(node_idx, emb_pad, wa, wb, b1):
  n = node_idx.shape[0]
  t = 2000
  grid = n // t

  def body(idx_ref, emb_ref, wa_ref, wb_ref, b_ref, ns_ref, a_ref, bm_ref):
    cols = lax.broadcasted_iota(jnp.int32, (1, HIDDEN), 1)
    onehot = (idx_ref[...] == cols).astype(jnp.float32)
    ns = jnp.dot(onehot, emb_ref[...], preferred_element_type=jnp.float32)
    ns_ref[...] = ns
    a_ref[...] = jnp.dot(ns, wa_ref[...], preferred_element_type=jnp.float32)
    bm_ref[...] = jnp.dot(ns, wb_ref[...],
                          preferred_element_type=jnp.float32) + b_ref[...]

  return pl.pallas_call(
      body,
      grid=(grid,),
      in_specs=[
          pl.BlockSpec((t, 1), lambda i: (i, 0)),
          pl.BlockSpec((HIDDEN, HIDDEN), lambda i: (0, 0)),
          pl.BlockSpec((HIDDEN, HIDDEN), lambda i: (0, 0)),
          pl.BlockSpec((HIDDEN, HIDDEN), lambda i: (0, 0)),
          pl.BlockSpec((1, HIDDEN), lambda i: (0, 0)),
      ],
      out_specs=(pl.BlockSpec((t, HIDDEN), lambda i: (i, 0)),
                 pl.BlockSpec((t, HIDDEN), lambda i: (i, 0)),
                 pl.BlockSpec((t, HIDDEN), lambda i: (i, 0))),
      out_shape=(jax.ShapeDtypeStruct((n, HIDDEN), jnp.float32),
                 jax.ShapeDtypeStruct((n, HIDDEN), jnp.float32),
                 jax.ShapeDtypeStruct((n, HIDDEN), jnp.float32)),
  )(node_idx.reshape(n, 1), emb_pad, wa, wb, b1.reshape(1, HIDDEN))


def _tc_edge(g, feat, we_pad, be, w2, b2, sc=None):
  e = g.shape[0]
  t = next(tt for tt in (2048, 2000, 1024, 512, 256) if e % tt == 0)
  grid = e // t
  inv2s2 = 1.0 / (2.0 * GAUSS_STEP * GAUSS_STEP)
  first = sc is None

  def body(g_ref, f_ref, we_ref, be_ref, w2_ref, b2_ref, *rest):
    if first:
      m_ref, sc_ref = rest
    else:
      sc_in, m_ref = rest
    x = f_ref[...]
    mu = lax.broadcasted_iota(jnp.int32, (1, HIDDEN), 1).astype(
        jnp.float32) * GAUSS_STEP
    ex = jnp.exp(-((x - mu) ** 2) * inv2s2)
    if first:
      cut = 1.0 / (1.0 + jnp.exp(5.0 * (x - (CUTOFF - 1.5))))
      sc_ref[...] = cut
    else:
      cut = sc_in[...]
    gate = _ssp(jnp.dot(ex, we_ref[...], preferred_element_type=jnp.float32)
                + be_ref[...]) * cut
    h = _ssp(g_ref[...])
    m_ref[...] = (jnp.dot(h, w2_ref[...], preferred_element_type=jnp.float32)
                  + b2_ref[...]) * gate

  in_specs = [
      pl.BlockSpec((t, HIDDEN), lambda i: (i, 0)),
      pl.BlockSpec((t, 1), lambda i: (i, 0)),
      pl.BlockSpec((HIDDEN, HIDDEN), lambda i: (0, 0)),
      pl.BlockSpec((1, HIDDEN), lambda i: (0, 0)),
      pl.BlockSpec((HIDDEN, HIDDEN), lambda i: (0, 0)),
      pl.BlockSpec((1, HIDDEN), lambda i: (0, 0)),
  ]
  args = [g, feat, we_pad, be.reshape(1, HIDDEN), w2, b2.reshape(1, HIDDEN)]
  m_spec = pl.BlockSpec((t, HIDDEN), lambda i: (i, 0))
  m_shape = jax.ShapeDtypeStruct((e, HIDDEN), jnp.float32)
  sc_spec = pl.BlockSpec((t, 1), lambda i: (i, 0))
  if first:
    out_specs = (m_spec, sc_spec)
    out_shape = (m_shape, jax.ShapeDtypeStruct((e, 1), jnp.float32))
  else:
    in_specs.append(sc_spec)
    args.append(sc)
    out_specs = m_spec
    out_shape = m_shape

  return pl.pallas_call(
      body,
      grid=(grid,),
      in_specs=in_specs,
      out_specs=out_specs,
      out_shape=out_shape,
  )(*args)


def _tc_node_post(ns, plist, w1, b1, w2, b2, nxt=None):
  n = ns.shape[0]
  t = 2000
  grid = n // t
  fused = nxt is not None
  np_ = len(plist)

  def body(ns_ref, *rest):
    p_refs = rest[:np_]
    w1_ref, b1_ref, w2_ref, b2_ref = rest[np_:np_ + 4]
    rest = rest[np_ + 4:]
    if fused:
      wa_ref, wb_ref, nb_ref, o_ref, a_ref, bm_ref = rest
    else:
      (o_ref,) = rest
    ms = p_refs[0][...]
    for pr in p_refs[1:]:
      ms = ms + pr[...]
    tt = _ssp(jnp.dot(ms, w1_ref[...], preferred_element_type=jnp.float32)
              + b1_ref[...])
    new = ns_ref[...] + jnp.dot(
        tt, w2_ref[...], preferred_element_type=jnp.float32) + b2_ref[...]
    o_ref[...] = new
    if fused:
      a_ref[...] = jnp.dot(new, wa_ref[...], preferred_element_type=jnp.float32)
      bm_ref[...] = jnp.dot(new, wb_ref[...],
                            preferred_element_type=jnp.float32) + nb_ref[...]

  in_specs = [pl.BlockSpec((t, HIDDEN), lambda i: (i, 0))]
  in_specs += [pl.BlockSpec((t, HIDDEN), lambda i: (i, 0))] * np_
  in_specs += [
      pl.BlockSpec((HIDDEN, HIDDEN), lambda i: (0, 0)),
      pl.BlockSpec((1, HIDDEN), lambda i: (0, 0)),
      pl.BlockSpec((HIDDEN, HIDDEN), lambda i: (0, 0)),
      pl.BlockSpec((1, HIDDEN), lambda i: (0, 0)),
  ]
  args = [ns] + list(plist) + [w1, b1.reshape(1, HIDDEN), w2,
                               b2.reshape(1, HIDDEN)]
  out_specs = pl.BlockSpec((t, HIDDEN), lambda i: (i, 0))
  out_shape = jax.ShapeDtypeStruct((n, HIDDEN), jnp.float32)
  if fused:
    wa, wb, nb1 = nxt
    in_specs += [
        pl.BlockSpec((HIDDEN, HIDDEN), lambda i: (0, 0)),
        pl.BlockSpec((HIDDEN, HIDDEN), lambda i: (0, 0)),
        pl.BlockSpec((1, HIDDEN), lambda i: (0, 0)),
    ]
    args += [wa, wb, nb1.reshape(1, HIDDEN)]
    out_specs = (out_specs,) * 3
    out_shape = (out_shape,) * 3

  return pl.pallas_call(
      body,
      grid=(grid,),
      in_specs=in_specs,
      out_specs=out_specs,
      out_shape=out_shape,
  )(*args)


def kernel(nodes, num_nodes, atom_edges, num_atom_edges, atom_edges_features,
           atom_embeddings, params):
  del num_nodes, num_atom_edges
  n_nodes = nodes.shape[1]
  node_idx = nodes[0].astype(jnp.int32)
  src = atom_edges[0, :, 0].astype(jnp.int32)
  dst = atom_edges[0, :, 1].astype(jnp.int32)
  feat = atom_edges_features[0].astype(jnp.float32)

  n_pad = ((n_nodes + CH * NS - 1) // (CH * NS)) * (CH * NS)
  emb_pad = jnp.zeros((HIDDEN, HIDDEN), jnp.float32).at[
      :atom_embeddings.shape[0]].set(atom_embeddings.astype(jnp.float32))

  e = src.shape[0]
  nsplit = 2
  unit = e // (256 * nsplit) * 256
  offs = [0] + [unit * (h + 1) for h in range(nsplit - 1)] + [e]
  srcs = [src[offs[h]:offs[h + 1]] for h in range(nsplit)]
  dsts = [dst[offs[h]:offs[h + 1]] for h in range(nsplit)]
  feats = [feat[offs[h]:offs[h + 1]] for h in range(nsplit)]

  nl = len(params)
  p = params[0]
  ns, a, bm = _tc_embed_pre(node_idx, emb_pad, p['W_node1'][:HIDDEN],
                            p['W_node1'][HIDDEN:], p['b_node1'])
  soft_cut = [None] * nsplit
  outs = []
  for li in range(nl):
    p = params[li]
    we_pad = jnp.zeros((HIDDEN, HIDDEN), jnp.float32).at[
        :p['W_edge'].shape[0]].set(p['W_edge'])
    g = [_sc_gather_add(a, bm, srcs[h], dsts[h]) for h in range(nsplit)]
    parts = []
    for h in range(nsplit):
      if soft_cut[h] is None:
        m, soft_cut[h] = _tc_edge(g[h], feats[h], we_pad, p['b_edge'],
                                  p['W_node2'], p['b_node2'])
      else:
        m = _tc_edge(g[h], feats[h], we_pad, p['b_edge'], p['W_node2'],
                     p['b_node2'], sc=soft_cut[h])
      parts.append(_sc_scatter(m, dsts[h], n_pad))
    plist = [pp[c, :n_nodes] for pp in parts for c in range(NC)]
    if li + 1 < nl:
      pn = params[li + 1]
      ns, a, bm = _tc_node_post(
          ns, plist, p['W_st1'], p['b_st1'], p['W_st2'], p['b_st2'],
          nxt=(pn['W_node1'][:HIDDEN], pn['W_node1'][HIDDEN:], pn['b_node1']))
    else:
      ns = _tc_node_post(ns, plist, p['W_st1'], p['b_st1'], p['W_st2'],
                         p['b_st2'])
    outs.append(ns)
  return jnp.stack(outs, axis=0)
